# Initial kernel scaffold; baseline (speedup 1.0000x reference)
import jax
import jax.numpy as jnp
from jax.experimental import pallas as pl


def kernel(fused_embeds, logits, entropy_memo, embed_memo):
    raise NotImplementedError("write your pallas kernel here")


# speedup vs baseline: 1.3075x; 1.3075x over previous
import jax
import jax.numpy as jnp
from jax import lax
from jax.experimental import pallas as pl

B = 4096
D = 512
M = 512
CHUNK = 512


def _body(logits_col_ref, logits_row_ref, vmemo_row_ref, vmemo_col_ref,
          fused_ref, emcat_ref, memo_pred_ref, pred_ref, ent_ref):
    lg = logits_col_ref[...]
    mx = jnp.max(lg, axis=1, keepdims=True)
    z = lg - mx
    ez = jnp.exp(z)
    s = jnp.sum(ez, axis=1, keepdims=True)
    p = ez / s
    logp = z - jnp.log(s)
    ent_col = -jnp.sum(p * logp, axis=1, keepdims=True)
    pred_ref[...] = p
    ent_ref[...] = ent_col
    y1_col = p[:, 1:2] > p[:, 0:1]

    l0 = logits_row_ref[0:1, :]
    l1 = logits_row_ref[1:2, :]
    mr = jnp.maximum(l0, l1)
    z0 = l0 - mr
    z1 = l1 - mr
    e0 = jnp.exp(z0)
    e1 = jnp.exp(z1)
    sr = e0 + e1
    p0 = e0 / sr
    p1 = e1 / sr
    lr = jnp.log(sr)
    ent_row = -(p0 * (z0 - lr) + p1 * (z1 - lr))
    y1_row = p1 > p0

    inf = jnp.float32(jnp.inf)
    iota_col_b = lax.broadcasted_iota(jnp.int32, (B, 1), 0)
    iota_row_m = lax.broadcasted_iota(jnp.int32, (1, M), 1)
    iota_col_m = lax.broadcasted_iota(jnp.int32, (M, 1), 0)

    sel_rows = []
    keep_rows = []
    for lbl in range(2):
        want = y1_row if lbl == 1 else jnp.logical_not(y1_row)
        want_col = y1_col if lbl == 1 else jnp.logical_not(y1_col)
        e_row = jnp.where(want, ent_row, inf)
        e_col = jnp.where(want_col, ent_col, inf)
        v_row = vmemo_row_ref[lbl:lbl + 1, :]
        v_col = vmemo_col_ref[:, lbl:lbl + 1]

        gt = v_col > v_row
        tie = jnp.logical_and(v_col == v_row, iota_col_m < iota_row_m)
        q = jnp.sum(jnp.where(jnp.logical_or(gt, tie), 1.0, 0.0),
                    axis=0, keepdims=True)
        c = jnp.sum(jnp.where(e_col < v_row, 1.0, 0.0),
                    axis=0, keepdims=True)
        keep_rows.append(jnp.where(c > q, 0.0, 1.0))

        sel_chunks = []
        for c0 in range(0, B, CHUNK):
            e_rc = e_row[:, c0:c0 + CHUNK]
            idx_rc = lax.broadcasted_iota(jnp.int32, (1, CHUNK), 1) + c0
            less = e_col < e_rc
            tiec = jnp.logical_and(e_col == e_rc, iota_col_b < idx_rc)
            r = jnp.sum(jnp.where(jnp.logical_or(less, tiec), 1.0, 0.0),
                        axis=0, keepdims=True)
            g = jnp.sum(jnp.where(v_col > e_rc, 1.0, 0.0),
                        axis=0, keepdims=True)
            sel_chunks.append(
                jnp.where(jnp.logical_and(r < float(M), g > r), 1.0, 0.0))
        sel_rows.append(jnp.concatenate(sel_chunks, axis=1))

    zero_b = jnp.zeros((1, B), jnp.float32)
    sel_mat = jnp.concatenate(
        sel_rows + [zero_b] * 6, axis=0)
    zero_m = jnp.zeros((1, M), jnp.float32)
    keep_mat = jnp.concatenate([
        jnp.concatenate([keep_rows[0], zero_m], axis=1),
        jnp.concatenate([zero_m, keep_rows[1]], axis=1),
    ] + [jnp.zeros((1, 2 * M), jnp.float32)] * 6, axis=0)

    fused = fused_ref[...].astype(jnp.bfloat16).astype(jnp.float32)
    emcat = emcat_ref[...].astype(jnp.bfloat16).astype(jnp.float32)
    hi = lax.Precision.HIGHEST
    cand_sum = jnp.dot(sel_mat, fused, precision=hi,
                       preferred_element_type=jnp.float32)
    keep_sum = jnp.dot(keep_mat, emcat, precision=hi,
                       preferred_element_type=jnp.float32)
    s_mat = cand_sum + keep_sum

    cos = lax.dot_general(fused, s_mat, (((1,), (1,)), ((), ())),
                          precision=hi,
                          preferred_element_type=jnp.float32)
    c0 = cos[:, 0:1]
    c1 = cos[:, 1:2]
    cm = jnp.maximum(c0, c1)
    ex0 = jnp.exp(c0 - cm)
    ex1 = jnp.exp(c1 - cm)
    den = ex0 + ex1
    memo_pred_ref[...] = jnp.concatenate([ex0 / den, ex1 / den], axis=1)


@jax.jit
def kernel(fused_embeds, logits, entropy_memo, embed_memo):
    memo_pred, pred, ent = pl.pallas_call(
        _body,
        out_shape=[
            jax.ShapeDtypeStruct((B, 2), jnp.float32),
            jax.ShapeDtypeStruct((B, 2), jnp.float32),
            jax.ShapeDtypeStruct((B, 1), jnp.float32),
        ],
    )(logits, logits.T, entropy_memo, entropy_memo.T,
      fused_embeds, embed_memo.reshape(2 * M, D))
    return memo_pred, pred, ent.reshape(B)


# speedup vs baseline: 1.7227x; 1.3175x over previous
import jax
import jax.numpy as jnp
from jax import lax
from jax.experimental import pallas as pl

B = 4096
D = 512
M = 512
LBIT = 1 << 30
SMASK = 0x7FFFFFFF


def _entropy_parts(l0, l1):
    m = jnp.maximum(l0, l1)
    z0 = l0 - m
    z1 = l1 - m
    e0 = jnp.exp(z0)
    e1 = jnp.exp(z1)
    s = e0 + e1
    p0 = e0 / s
    p1 = e1 / s
    ls = jnp.log(s)
    ent = -(p0 * (z0 - ls) + p1 * (z1 - ls))
    return ent, p0, p1


def _keys(l0, l1):
    ent, p0, p1 = _entropy_parts(l0, l1)
    bits = lax.bitcast_convert_type(ent, jnp.int32) & SMASK
    y1 = p1 > p0
    return jnp.where(y1, bits | LBIT, bits), ent, p0, p1


def _body(logits_col_ref, l0_blk_ref, l1_blk_ref, logits_row_ref,
          vmemo_row_ref, vmemo_col_ref, fused_ref, emcat_ref,
          memo_pred_ref, pred_ref, ent_ref):
    lg = logits_col_ref[...]
    keys_col, ent_col, p0c, p1c = _keys(lg[:, 0:1], lg[:, 1:2])
    pred_ref[...] = jnp.concatenate([p0c, p1c], axis=1)
    ent_ref[...] = ent_col
    keys_blk, _, _, _ = _keys(l0_blk_ref[...], l1_blk_ref[...])
    keys_row, _, _, _ = _keys(logits_row_ref[0:1, :],
                              logits_row_ref[1:2, :])

    count0 = jnp.sum(jnp.where(keys_blk < LBIT, 1, 0))

    def mbits(x):
        return lax.bitcast_convert_type(x, jnp.int32) & SMASK
    mk_row = jnp.concatenate(
        [mbits(vmemo_row_ref[0:1, :]),
         mbits(vmemo_row_ref[1:2, :]) | LBIT], axis=1)
    mk_col = jnp.concatenate(
        [mbits(vmemo_col_ref[:, 0:1]),
         mbits(vmemo_col_ref[:, 1:2]) | LBIT], axis=0)

    iota_row_2m = lax.broadcasted_iota(jnp.int32, (1, 2 * M), 1)
    iota_col_2m = lax.broadcasted_iota(jnp.int32, (2 * M, 1), 0)
    gt = mk_col > mk_row
    tie = jnp.logical_and(mk_col == mk_row, iota_col_2m < iota_row_2m)
    q = jnp.sum(jnp.where(jnp.logical_or(gt, tie), 1, 0),
                axis=0, keepdims=True)
    c = jnp.sum(jnp.where(keys_col < mk_row, 1, 0),
                axis=0, keepdims=True)
    lane_hi = iota_row_2m >= M
    q = q - jnp.where(lane_hi, 0, M)
    c = c - jnp.where(lane_hi, count0, 0)
    replaced = c > q
    keep_f = jnp.where(replaced, 0.0, 1.0)
    rep1 = jnp.where(replaced, 1, 0)
    k0 = jnp.sum(jnp.where(lane_hi, 0, rep1))
    k1 = jnp.sum(jnp.where(lane_hi, rep1, 0))
    t0 = k0
    t1 = count0 + k1

    def vstep(i, cur):
        cur0, cur1 = cur
        bit = lax.shift_left(jnp.int32(1), 30 - i)
        try0 = cur0 + bit
        try1 = cur1 + bit
        n0 = jnp.sum(jnp.where(keys_blk < try0, 1, 0))
        n1 = jnp.sum(jnp.where(keys_blk < try1, 1, 0))
        return (jnp.where(n0 <= t0, try0, cur0),
                jnp.where(n1 <= t1, try1, cur1))
    v0, v1 = lax.fori_loop(0, 31, vstep, (jnp.int32(0), jnp.int32(0)))
    s0 = t0 - jnp.sum(jnp.where(keys_blk < v0, 1, 0))
    s1 = t1 - jnp.sum(jnp.where(keys_blk < v1, 1, 0))

    iota_blk = (lax.broadcasted_iota(jnp.int32, (32, 128), 0) * 128
                + lax.broadcasted_iota(jnp.int32, (32, 128), 1))
    lab0_blk = keys_blk < LBIT
    eq0 = jnp.logical_and(keys_blk == v0, lab0_blk)
    eq1 = jnp.logical_and(keys_blk == v1, jnp.logical_not(lab0_blk))

    def istep(i, cur):
        cur0, cur1 = cur
        bit = lax.shift_left(jnp.int32(1), 12 - i)
        try0 = cur0 + bit
        try1 = cur1 + bit
        n0 = jnp.sum(jnp.where(jnp.logical_and(eq0, iota_blk < try0), 1, 0))
        n1 = jnp.sum(jnp.where(jnp.logical_and(eq1, iota_blk < try1), 1, 0))
        return (jnp.where(n0 <= s0, try0, cur0),
                jnp.where(n1 <= s1, try1, cur1))
    i0, i1 = lax.fori_loop(0, 13, istep, (jnp.int32(0), jnp.int32(0)))

    iota_row_b = lax.broadcasted_iota(jnp.int32, (1, B), 1)
    lab0_row = keys_row < LBIT
    sel0 = jnp.logical_and(
        lab0_row,
        jnp.logical_or(keys_row < v0,
                       jnp.logical_and(keys_row == v0, iota_row_b < i0)))
    sel1 = jnp.logical_and(
        jnp.logical_not(lab0_row),
        jnp.logical_or(keys_row < v1,
                       jnp.logical_and(keys_row == v1, iota_row_b < i1)))
    zero_b = jnp.zeros((1, B), jnp.float32)
    sel_mat = jnp.concatenate(
        [jnp.where(sel0, 1.0, 0.0), jnp.where(sel1, 1.0, 0.0)]
        + [zero_b] * 6, axis=0)

    zero_2m = jnp.zeros((1, 2 * M), jnp.float32)
    keep_mat = jnp.concatenate(
        [jnp.where(lane_hi, 0.0, keep_f), jnp.where(lane_hi, keep_f, 0.0)]
        + [zero_2m] * 6, axis=0)

    fused = fused_ref[...].astype(jnp.bfloat16).astype(jnp.float32)
    emcat = emcat_ref[...].astype(jnp.bfloat16).astype(jnp.float32)
    hi = lax.Precision.HIGHEST
    cand_sum = jnp.dot(sel_mat, fused, precision=hi,
                       preferred_element_type=jnp.float32)
    keep_sum = jnp.dot(keep_mat, emcat, precision=hi,
                       preferred_element_type=jnp.float32)
    s_mat = cand_sum + keep_sum

    cos = lax.dot_general(fused, s_mat, (((1,), (1,)), ((), ())),
                          precision=hi,
                          preferred_element_type=jnp.float32)
    c0 = cos[:, 0:1]
    c1 = cos[:, 1:2]
    cm = jnp.maximum(c0, c1)
    ex0 = jnp.exp(c0 - cm)
    ex1 = jnp.exp(c1 - cm)
    den = ex0 + ex1
    memo_pred_ref[...] = jnp.concatenate([ex0 / den, ex1 / den], axis=1)


@jax.jit
def kernel(fused_embeds, logits, entropy_memo, embed_memo):
    l0_blk = logits[:, 0].reshape(32, 128)
    l1_blk = logits[:, 1].reshape(32, 128)
    memo_pred, pred, ent = pl.pallas_call(
        _body,
        out_shape=[
            jax.ShapeDtypeStruct((B, 2), jnp.float32),
            jax.ShapeDtypeStruct((B, 2), jnp.float32),
            jax.ShapeDtypeStruct((B, 1), jnp.float32),
        ],
    )(logits, l0_blk, l1_blk, logits.T, entropy_memo, entropy_memo.T,
      fused_embeds, embed_memo.reshape(2 * M, D))
    return memo_pred, pred, ent.reshape(B)


# speedup vs baseline: 1.8875x; 1.0957x over previous
import jax
import jax.numpy as jnp
from jax import lax
from jax.experimental import pallas as pl

B = 4096
D = 512
M = 512
LBIT = 1 << 30
SMASK = 0x7FFFFFFF


def _entropy_parts(l0, l1):
    m = jnp.maximum(l0, l1)
    z0 = l0 - m
    z1 = l1 - m
    e0 = jnp.exp(z0)
    e1 = jnp.exp(z1)
    s = e0 + e1
    p0 = e0 / s
    p1 = e1 / s
    ls = jnp.log(s)
    ent = -(p0 * (z0 - ls) + p1 * (z1 - ls))
    return ent, p0, p1


def _keys(l0, l1):
    ent, p0, p1 = _entropy_parts(l0, l1)
    bits = lax.bitcast_convert_type(ent, jnp.int32) & SMASK
    y1 = p1 > p0
    return jnp.where(y1, bits | LBIT, bits), ent, p0, p1


def _body(logits_col_ref, logits_row_ref,
          vmemo_row_ref, vmemo_col_ref, fused_ref, emcat_ref,
          memo_pred_ref, pred_ref, ent_ref):
    lg = logits_col_ref[...]
    keys_col, ent_col, p0c, p1c = _keys(lg[:, 0:1], lg[:, 1:2])
    pred_ref[...] = jnp.concatenate([p0c, p1c], axis=1)
    ent_ref[...] = ent_col
    keys_row, _, _, _ = _keys(logits_row_ref[0:1, :],
                              logits_row_ref[1:2, :])

    count0 = jnp.sum(jnp.where(keys_col < LBIT, 1, 0))

    def mbits(x):
        return lax.bitcast_convert_type(x, jnp.int32) & SMASK
    mk_row = jnp.concatenate(
        [mbits(vmemo_row_ref[0:1, :]),
         mbits(vmemo_row_ref[1:2, :]) | LBIT], axis=1)
    mk_col = jnp.concatenate(
        [mbits(vmemo_col_ref[:, 0:1]),
         mbits(vmemo_col_ref[:, 1:2]) | LBIT], axis=0)

    iota_row_2m = lax.broadcasted_iota(jnp.int32, (1, 2 * M), 1)
    iota_col_2m = lax.broadcasted_iota(jnp.int32, (2 * M, 1), 0)
    gt = mk_col > mk_row
    tie = jnp.logical_and(mk_col == mk_row, iota_col_2m < iota_row_2m)
    q = jnp.sum(jnp.where(jnp.logical_or(gt, tie), 1, 0),
                axis=0, keepdims=True)
    c = jnp.sum(jnp.where(keys_col < mk_row, 1, 0),
                axis=0, keepdims=True)
    lane_hi = iota_row_2m >= M
    q = q - jnp.where(lane_hi, 0, M)
    c = c - jnp.where(lane_hi, count0, 0)
    replaced = c > q
    keep_f = jnp.where(replaced, 0.0, 1.0)
    rep1 = jnp.where(replaced, 1, 0)
    k0 = jnp.sum(jnp.where(lane_hi, 0, rep1))
    k1 = jnp.sum(jnp.where(lane_hi, rep1, 0))
    t0 = k0
    t1 = count0 + k1

    cur0 = jnp.int32(0)
    cur1 = jnp.int32(0)
    for shift, width in ((24, 128), (16, 256), (8, 256), (0, 256)):
        jj = lax.broadcasted_iota(jnp.int32, (1, width), 1) * (1 << shift)
        thr0 = cur0 + jj
        thr1 = cur1 + jj
        n0 = jnp.sum(jnp.where(keys_col < thr0, 1, 0), axis=0, keepdims=True)
        n1 = jnp.sum(jnp.where(keys_col < thr1, 1, 0), axis=0, keepdims=True)
        j0 = jnp.sum(jnp.where(n0 <= t0, 1, 0)) - 1
        j1 = jnp.sum(jnp.where(n1 <= t1, 1, 0)) - 1
        cur0 = cur0 + lax.shift_left(j0, shift)
        cur1 = cur1 + lax.shift_left(j1, shift)
    v0, v1 = cur0, cur1
    s0 = t0 - jnp.sum(jnp.where(keys_col < v0, 1, 0))
    s1 = t1 - jnp.sum(jnp.where(keys_col < v1, 1, 0))

    iota_col_b = lax.broadcasted_iota(jnp.int32, (B, 1), 0)
    lab0_col = keys_col < LBIT
    eq0 = jnp.logical_and(keys_col == v0, lab0_col)
    eq1 = jnp.logical_and(keys_col == v1, jnp.logical_not(lab0_col))
    i0 = jnp.int32(0)
    i1 = jnp.int32(0)
    for shift, width in ((7, 64), (0, 128)):
        jj = lax.broadcasted_iota(jnp.int32, (1, width), 1) * (1 << shift)
        thr0 = i0 + jj
        thr1 = i1 + jj
        n0 = jnp.sum(jnp.where(jnp.logical_and(eq0, iota_col_b < thr0), 1, 0),
                     axis=0, keepdims=True)
        n1 = jnp.sum(jnp.where(jnp.logical_and(eq1, iota_col_b < thr1), 1, 0),
                     axis=0, keepdims=True)
        j0 = jnp.sum(jnp.where(n0 <= s0, 1, 0)) - 1
        j1 = jnp.sum(jnp.where(n1 <= s1, 1, 0)) - 1
        i0 = i0 + lax.shift_left(j0, shift)
        i1 = i1 + lax.shift_left(j1, shift)

    iota_row_b = lax.broadcasted_iota(jnp.int32, (1, B), 1)
    lab0_row = keys_row < LBIT
    sel0 = jnp.logical_and(
        lab0_row,
        jnp.logical_or(keys_row < v0,
                       jnp.logical_and(keys_row == v0, iota_row_b < i0)))
    sel1 = jnp.logical_and(
        jnp.logical_not(lab0_row),
        jnp.logical_or(keys_row < v1,
                       jnp.logical_and(keys_row == v1, iota_row_b < i1)))
    zero_b = jnp.zeros((1, B), jnp.float32)
    sel_mat = jnp.concatenate(
        [jnp.where(sel0, 1.0, 0.0), jnp.where(sel1, 1.0, 0.0)]
        + [zero_b] * 6, axis=0)

    zero_2m = jnp.zeros((1, 2 * M), jnp.float32)
    keep_mat = jnp.concatenate(
        [jnp.where(lane_hi, 0.0, keep_f), jnp.where(lane_hi, keep_f, 0.0)]
        + [zero_2m] * 6, axis=0)

    fused = fused_ref[...].astype(jnp.bfloat16).astype(jnp.float32)
    emcat = emcat_ref[...].astype(jnp.bfloat16).astype(jnp.float32)
    hi = lax.Precision.HIGHEST
    cand_sum = jnp.dot(sel_mat, fused,
                       preferred_element_type=jnp.float32)
    keep_sum = jnp.dot(keep_mat, emcat,
                       preferred_element_type=jnp.float32)
    s_mat = cand_sum + keep_sum

    cos = lax.dot_general(fused, s_mat, (((1,), (1,)), ((), ())),
                          precision=hi,
                          preferred_element_type=jnp.float32)
    c0 = cos[:, 0:1]
    c1 = cos[:, 1:2]
    cm = jnp.maximum(c0, c1)
    ex0 = jnp.exp(c0 - cm)
    ex1 = jnp.exp(c1 - cm)
    den = ex0 + ex1
    memo_pred_ref[...] = jnp.concatenate([ex0 / den, ex1 / den], axis=1)


@jax.jit
def kernel(fused_embeds, logits, entropy_memo, embed_memo):
    memo_pred, pred, ent = pl.pallas_call(
        _body,
        out_shape=[
            jax.ShapeDtypeStruct((B, 2), jnp.float32),
            jax.ShapeDtypeStruct((B, 2), jnp.float32),
            jax.ShapeDtypeStruct((B, 1), jnp.float32),
        ],
    )(logits, logits.T, entropy_memo, entropy_memo.T,
      fused_embeds, embed_memo.reshape(2 * M, D))
    return memo_pred, pred, ent.reshape(B)


# speedup vs baseline: 1.8899x; 1.0013x over previous
import jax
import jax.numpy as jnp
from jax import lax
from jax.experimental import pallas as pl

B = 4096
D = 512
M = 512
LBIT = 1 << 30
SMASK = 0x7FFFFFFF


def _entropy_parts(l0, l1):
    m = jnp.maximum(l0, l1)
    z0 = l0 - m
    z1 = l1 - m
    e0 = jnp.exp(z0)
    e1 = jnp.exp(z1)
    s = e0 + e1
    p0 = e0 / s
    p1 = e1 / s
    ls = jnp.log(s)
    ent = -(p0 * (z0 - ls) + p1 * (z1 - ls))
    return ent, p0, p1


def _keys(l0, l1):
    ent, p0, p1 = _entropy_parts(l0, l1)
    bits = lax.bitcast_convert_type(ent, jnp.int32) & SMASK
    y1 = p1 > p0
    return jnp.where(y1, bits | LBIT, bits), ent, p0, p1


def _body(logits_col_ref, logits_row_ref,
          vmemo_row_ref, vmemo_col_ref, fused_ref, emcat_ref,
          memo_pred_ref, pred_ref, ent_ref):
    lg = logits_col_ref[...]
    keys_col, ent_col, p0c, p1c = _keys(lg[:, 0:1], lg[:, 1:2])
    pred_ref[...] = jnp.concatenate([p0c, p1c], axis=1)
    ent_ref[...] = ent_col
    keys_row, _, _, _ = _keys(logits_row_ref[0:1, :],
                              logits_row_ref[1:2, :])

    count0 = jnp.sum(jnp.where(keys_row < LBIT, 1, 0))

    def mbits(x):
        return lax.bitcast_convert_type(x, jnp.int32) & SMASK
    mk_row = jnp.concatenate(
        [mbits(vmemo_row_ref[0:1, :]),
         mbits(vmemo_row_ref[1:2, :]) | LBIT], axis=1)
    mk_col = jnp.concatenate(
        [mbits(vmemo_col_ref[:, 0:1]),
         mbits(vmemo_col_ref[:, 1:2]) | LBIT], axis=0)

    iota_row_2m = lax.broadcasted_iota(jnp.int32, (1, 2 * M), 1)
    iota_col_2m = lax.broadcasted_iota(jnp.int32, (2 * M, 1), 0)
    gt = mk_col > mk_row
    tie = jnp.logical_and(mk_col == mk_row, iota_col_2m < iota_row_2m)
    q = jnp.sum(jnp.where(jnp.logical_or(gt, tie), 1, 0),
                axis=0, keepdims=True)
    c = jnp.sum(jnp.where(keys_col < mk_row, 1, 0),
                axis=0, keepdims=True)
    lane_hi = iota_row_2m >= M
    q = q - jnp.where(lane_hi, 0, M)
    c = c - jnp.where(lane_hi, count0, 0)
    replaced = c > q
    keep_f = jnp.where(replaced, 0.0, 1.0)
    rep1 = jnp.where(replaced, 1, 0)
    k0 = jnp.sum(jnp.where(lane_hi, 0, rep1))
    k1 = jnp.sum(jnp.where(lane_hi, rep1, 0))
    t0 = k0
    t1 = count0 + k1

    cur0 = jnp.int32(0)
    cur1 = jnp.int32(0)
    for shift, width in ((24, 128), (16, 256), (8, 256), (0, 256)):
        jj = lax.broadcasted_iota(jnp.int32, (1, width), 1) * (1 << shift)
        thr0 = cur0 + jj
        thr1 = cur1 + jj
        n0 = jnp.sum(jnp.where(keys_col < thr0, 1, 0), axis=0, keepdims=True)
        n1 = jnp.sum(jnp.where(keys_col < thr1, 1, 0), axis=0, keepdims=True)
        j0 = jnp.sum(jnp.where(n0 <= t0, 1, 0)) - 1
        j1 = jnp.sum(jnp.where(n1 <= t1, 1, 0)) - 1
        cur0 = cur0 + lax.shift_left(j0, shift)
        cur1 = cur1 + lax.shift_left(j1, shift)
    v0, v1 = cur0, cur1
    s0 = t0 - jnp.sum(jnp.where(keys_col < v0, 1, 0))
    s1 = t1 - jnp.sum(jnp.where(keys_col < v1, 1, 0))

    iota_col_b = lax.broadcasted_iota(jnp.int32, (B, 1), 0)
    lab0_col = keys_col < LBIT
    eq0 = jnp.logical_and(keys_col == v0, lab0_col)
    eq1 = jnp.logical_and(keys_col == v1, jnp.logical_not(lab0_col))
    i0 = jnp.int32(0)
    i1 = jnp.int32(0)
    for shift, width in ((7, 64), (0, 128)):
        jj = lax.broadcasted_iota(jnp.int32, (1, width), 1) * (1 << shift)
        thr0 = i0 + jj
        thr1 = i1 + jj
        n0 = jnp.sum(jnp.where(jnp.logical_and(eq0, iota_col_b < thr0), 1, 0),
                     axis=0, keepdims=True)
        n1 = jnp.sum(jnp.where(jnp.logical_and(eq1, iota_col_b < thr1), 1, 0),
                     axis=0, keepdims=True)
        j0 = jnp.sum(jnp.where(n0 <= s0, 1, 0)) - 1
        j1 = jnp.sum(jnp.where(n1 <= s1, 1, 0)) - 1
        i0 = i0 + lax.shift_left(j0, shift)
        i1 = i1 + lax.shift_left(j1, shift)

    iota_row_b = lax.broadcasted_iota(jnp.int32, (1, B), 1)
    lab0_row = keys_row < LBIT
    sel0 = jnp.logical_and(
        lab0_row,
        jnp.logical_or(keys_row < v0,
                       jnp.logical_and(keys_row == v0, iota_row_b < i0)))
    sel1 = jnp.logical_and(
        jnp.logical_not(lab0_row),
        jnp.logical_or(keys_row < v1,
                       jnp.logical_and(keys_row == v1, iota_row_b < i1)))
    zero_b = jnp.zeros((1, B), jnp.float32)
    sel_mat = jnp.concatenate(
        [jnp.where(sel0, 1.0, 0.0), jnp.where(sel1, 1.0, 0.0)]
        + [zero_b] * 6, axis=0)

    zero_2m = jnp.zeros((1, 2 * M), jnp.float32)
    keep_mat = jnp.concatenate(
        [jnp.where(lane_hi, 0.0, keep_f), jnp.where(lane_hi, keep_f, 0.0)]
        + [zero_2m] * 6, axis=0)

    fused = fused_ref[...].astype(jnp.bfloat16).astype(jnp.float32)
    emcat = emcat_ref[...].astype(jnp.bfloat16).astype(jnp.float32)
    hi = lax.Precision.HIGHEST
    cand_sum = jnp.dot(sel_mat, fused,
                       preferred_element_type=jnp.float32)
    keep_sum = jnp.dot(keep_mat, emcat,
                       preferred_element_type=jnp.float32)
    s_mat = cand_sum + keep_sum
    s_t = jnp.transpose(s_mat)

    cos = jnp.dot(fused, s_t, precision=hi,
                  preferred_element_type=jnp.float32)
    c0 = cos[:, 0:1]
    c1 = cos[:, 1:2]
    cm = jnp.maximum(c0, c1)
    ex0 = jnp.exp(c0 - cm)
    ex1 = jnp.exp(c1 - cm)
    den = ex0 + ex1
    memo_pred_ref[...] = jnp.concatenate([ex0 / den, ex1 / den], axis=1)


@jax.jit
def kernel(fused_embeds, logits, entropy_memo, embed_memo):
    memo_pred, pred, ent = pl.pallas_call(
        _body,
        out_shape=[
            jax.ShapeDtypeStruct((B, 2), jnp.float32),
            jax.ShapeDtypeStruct((B, 2), jnp.float32),
            jax.ShapeDtypeStruct((B, 1), jnp.float32),
        ],
    )(logits, logits.T, entropy_memo, entropy_memo.T,
      fused_embeds, embed_memo.reshape(2 * M, D))
    return memo_pred, pred, ent.reshape(B)
